# baseline (speedup 1.0000x reference)
import functools

import jax
import jax.numpy as jnp
from jax import lax
from jax.experimental import pallas as pl
from jax.experimental.pallas import tpu as pltpu
from jax.experimental.pallas import tpu_sc as plsc

NC = 2
NS = 16
L = 16


def _scores_kernel(B, D):
    NW = NC * NS
    BPW = B // NW

    mesh = plsc.VectorSubcoreMesh(core_axis_name="c", subcore_axis_name="s")

    @functools.partial(
        pl.kernel,
        out_type=jax.ShapeDtypeStruct((B,), jnp.float32),
        mesh=mesh,
        scratch_types=[
            pltpu.VMEM((BPW,), jnp.int32),
            pltpu.VMEM((BPW,), jnp.int32),
            pltpu.VMEM((BPW, D), jnp.float32),
            pltpu.VMEM((BPW, D), jnp.float32),
            pltpu.VMEM((BPW,), jnp.float32),
            pltpu.VMEM((BPW,), jnp.float32),
            pltpu.VMEM((L,), jnp.float32),
            pltpu.VMEM((BPW,), jnp.float32),
            pltpu.SemaphoreType.DMA,
        ],
        compiler_params=pltpu.CompilerParams(
            needs_layout_passes=False, use_tc_tiling_on_sc=False),
    )
    def run(uid_h, iid_h, ut_h, it_h, ubf_h, ibf_h, gb_h, out_h,
            uid_v, iid_v, urows_v, irows_v, ub_v, ib_v, gb_v, out_v, sem):
        wid = lax.axis_index("c") * NS + lax.axis_index("s")
        base = wid * BPW

        pltpu.sync_copy(uid_h.at[pl.ds(base, BPW)], uid_v)
        pltpu.sync_copy(iid_h.at[pl.ds(base, BPW)], iid_v)
        pltpu.sync_copy(gb_h, gb_v)

        cps = [
            pltpu.async_copy(ut_h.at[uid_v], urows_v, sem),
            pltpu.async_copy(it_h.at[iid_v], irows_v, sem),
            pltpu.async_copy(ubf_h.at[uid_v], ub_v, sem),
            pltpu.async_copy(ibf_h.at[iid_v], ib_v, sem),
        ]
        for cp in cps:
            cp.wait()

        iota = lax.iota(jnp.int32, L)
        gb = gb_v[...]

        def body(blk, carry):
            off = blk * L
            row16 = off + iota
            acc = jnp.zeros((L,), jnp.float32)
            for d in range(D):
                cold = jnp.full((L,), d, jnp.int32)
                ucol = plsc.load_gather(urows_v, [row16, cold])
                icol = plsc.load_gather(irows_v, [row16, cold])
                acc = acc + ucol * icol
            dots = acc + ub_v[pl.ds(off, L)] + ib_v[pl.ds(off, L)] + gb
            out_v[pl.ds(off, L)] = dots
            return carry

        lax.fori_loop(0, BPW // L, body, 0)

        pltpu.sync_copy(out_v, out_h.at[pl.ds(base, BPW)])

    return run


def kernel(user_ids, item_ids, user_table, item_table, user_bias, item_bias,
           global_bias):
    B = user_ids.shape[0]
    D = user_table.shape[1]
    run = _scores_kernel(B, D)
    gb16 = jnp.broadcast_to(
        global_bias.reshape(()).astype(jnp.float32), (L,))
    return run(
        user_ids.astype(jnp.int32),
        item_ids.astype(jnp.int32),
        user_table,
        item_table,
        user_bias.reshape(-1),
        item_bias.reshape(-1),
        gb16,
    )
